# Initial kernel scaffold; baseline (speedup 1.0000x reference)
import jax
import jax.numpy as jnp
from jax.experimental import pallas as pl


def kernel(de1, de2, f, W1, W2):
    raise NotImplementedError("write your pallas kernel here")


# speedup vs baseline: 2.1877x; 2.1877x over previous
import functools

import jax
import jax.numpy as jnp
from jax import lax
from jax.experimental import pallas as pl
from jax.experimental.pallas import tpu as pltpu
from jax.experimental.pallas import tpu_sc as plsc

E = 64
OUT_W = 2 * E + 1

GATHER = 128
MACRO = 1024
Q = 256


def _sc_body(rows_per_w, de1_hbm, de2_hbm, f_hbm, w1_hbm, w2_hbm,
             out_hbm, idx1_v, idx2_v, f_v, b1, b2, out_v, sem1, sem2):
    nc = 2
    wid = lax.axis_index("s") * nc + lax.axis_index("c")
    base = wid * rows_per_w
    n_macro = rows_per_w // MACRO
    lanes = lax.iota(jnp.int32, 16)

    def macro_body(i, _):
        row0 = pl.multiple_of(base + i * MACRO, MACRO)
        g0 = pl.multiple_of(row0 // GATHER, MACRO // GATHER)
        pltpu.sync_copy(de1_hbm.at[pl.ds(g0, MACRO // GATHER)], idx1_v)
        pltpu.sync_copy(de2_hbm.at[pl.ds(g0, MACRO // GATHER)], idx2_v)
        pltpu.sync_copy(f_hbm.at[pl.ds(row0, MACRO)], f_v)
        for q in range(MACRO // Q):
            for g in range(Q // GATHER):
                jj = q * (Q // GATHER) + g
                pltpu.async_copy(w1_hbm.at[idx1_v.at[jj]],
                                 b1.at[pl.ds(g * GATHER, GATHER)], sem1)
                pltpu.async_copy(w2_hbm.at[idx2_v.at[jj]],
                                 b2.at[pl.ds(g * GATHER, GATHER)], sem2)
            for g in range(Q // GATHER):
                jj = q * (Q // GATHER) + g
                pltpu.make_async_copy(w1_hbm.at[idx1_v.at[jj]],
                                      b1.at[pl.ds(g * GATHER, GATHER)],
                                      sem1).wait()
                pltpu.make_async_copy(w2_hbm.at[idx2_v.at[jj]],
                                      b2.at[pl.ds(g * GATHER, GATHER)],
                                      sem2).wait()

            def row_body(r, _):
                rb = r * OUT_W
                for k in range(E // 16):
                    plsc.store_scatter(out_v, [lanes + (rb + 16 * k)],
                                       b1[r, pl.ds(16 * k, 16)])
                for k in range(E // 16):
                    plsc.store_scatter(out_v, [lanes + (rb + E + 16 * k)],
                                       b2[r, pl.ds(16 * k, 16)])

            lax.fori_loop(0, Q, row_body, None)
            for j in range(Q // 16):
                addr = (lanes + 16 * j) * OUT_W + 2 * E
                plsc.store_scatter(out_v, [addr],
                                   f_v[pl.ds(q * Q + 16 * j, 16)])
            pltpu.sync_copy(
                out_v, out_hbm.at[pl.ds((row0 + q * Q) * OUT_W, Q * OUT_W)])

    lax.fori_loop(0, n_macro, macro_body, None)


def kernel(de1, de2, f, W1, W2):
    B, L = de1.shape
    n = B * L
    info = plsc.get_sparse_core_info()
    nw = info.num_cores * info.num_subcores
    rows_per_w = n // nw
    assert rows_per_w % MACRO == 0

    de1f = de1.reshape(n // GATHER, GATHER)
    de2f = de2.reshape(n // GATHER, GATHER)
    ff = f.reshape(n)
    W1p = jnp.pad(W1, ((0, 0), (0, 128 - E)))
    W2p = jnp.pad(W2, ((0, 0), (0, 128 - E)))

    mesh = plsc.VectorSubcoreMesh(core_axis_name="c", subcore_axis_name="s")
    run = pl.kernel(
        functools.partial(_sc_body, rows_per_w),
        out_type=jax.ShapeDtypeStruct((n * OUT_W,), jnp.float32),
        mesh=mesh,
        scratch_types=[
            pltpu.VMEM((MACRO // GATHER, GATHER), jnp.int32),
            pltpu.VMEM((MACRO // GATHER, GATHER), jnp.int32),
            pltpu.VMEM((MACRO,), jnp.float32),
            pltpu.VMEM((Q, 128), jnp.float32),
            pltpu.VMEM((Q, 128), jnp.float32),
            pltpu.VMEM((Q * OUT_W,), jnp.float32),
            pltpu.SemaphoreType.DMA,
            pltpu.SemaphoreType.DMA,
        ],
        compiler_params=pltpu.CompilerParams(needs_layout_passes=False),
    )
    out = run(de1f, de2f, ff, W1p, W2p)
    return out.reshape(B, L, OUT_W)


# speedup vs baseline: 2.5555x; 1.1681x over previous
import functools

import jax
import jax.numpy as jnp
from jax import lax
from jax.experimental import pallas as pl
from jax.experimental.pallas import tpu as pltpu
from jax.experimental.pallas import tpu_sc as plsc

E = 64
OUT_W = 2 * E + 1

STEP = 128
MACRO = 1024
NSTEP = MACRO // STEP
OUT_STEP = STEP * OUT_W


def _sc_body(rows_per_w, de1_hbm, de2_hbm, f_hbm, w1_hbm, w2_hbm,
             out_hbm, idx1_v, idx2_v, f_v, b1_0, b1_1, b2_0, b2_1,
             out_0, out_1, sem1, sem2, sem_o):
    nc = 2
    wid = lax.axis_index("s") * nc + lax.axis_index("c")
    base = wid * rows_per_w
    n_macro = rows_per_w // MACRO
    lanes = lax.iota(jnp.int32, 16)
    b1s, b2s, outs = (b1_0, b1_1), (b2_0, b2_1), (out_0, out_1)

    def gather(j, p):
        pltpu.async_copy(w1_hbm.at[idx1_v.at[j]], b1s[p], sem1)
        pltpu.async_copy(w2_hbm.at[idx2_v.at[j]], b2s[p], sem2)

    def gather_wait(j, p):
        pltpu.make_async_copy(w1_hbm.at[idx1_v.at[j]], b1s[p], sem1).wait()
        pltpu.make_async_copy(w2_hbm.at[idx2_v.at[j]], b2s[p], sem2).wait()

    def out_dst(row0, j):
        return out_hbm.at[pl.ds((row0 + j * STEP) * OUT_W, OUT_STEP)]

    def interleave(j, p):
        b1, b2, out_v = b1s[p], b2s[p], outs[p]

        def row_body(r, addr):
            for k in range(E // 16):
                plsc.store_scatter(out_v, [addr + 16 * k],
                                   b1[r, pl.ds(16 * k, 16)])
            for k in range(E // 16):
                plsc.store_scatter(out_v, [addr + (E + 16 * k)],
                                   b2[r, pl.ds(16 * k, 16)])
            return addr + OUT_W

        lax.fori_loop(0, STEP, row_body, lanes, unroll=2)
        for t in range(STEP // 16):
            addr = (lanes + 16 * t) * OUT_W + 2 * E
            plsc.store_scatter(out_v, [addr],
                               f_v[pl.ds(j * STEP + 16 * t, 16)])

    def macro_body(m, _):
        row0 = pl.multiple_of(base + m * MACRO, MACRO)
        g0 = pl.multiple_of(row0 // STEP, NSTEP)
        pltpu.sync_copy(de1_hbm.at[pl.ds(g0, NSTEP)], idx1_v)
        pltpu.sync_copy(de2_hbm.at[pl.ds(g0, NSTEP)], idx2_v)
        pltpu.sync_copy(f_hbm.at[pl.ds(row0, MACRO)], f_v)
        gather(0, 0)
        for j in range(NSTEP):
            p = j % 2
            if j + 1 < NSTEP:
                gather(j + 1, 1 - p)
            if j >= 2:
                pltpu.make_async_copy(outs[p], out_dst(row0, j - 2),
                                      sem_o).wait()
            gather_wait(j, p)
            interleave(j, p)
            pltpu.async_copy(outs[p], out_dst(row0, j), sem_o)
        for j in (NSTEP - 2, NSTEP - 1):
            pltpu.make_async_copy(outs[j % 2], out_dst(row0, j),
                                  sem_o).wait()

    lax.fori_loop(0, n_macro, macro_body, None)


def kernel(de1, de2, f, W1, W2):
    B, L = de1.shape
    n = B * L
    info = plsc.get_sparse_core_info()
    nw = info.num_cores * info.num_subcores
    rows_per_w = n // nw
    assert rows_per_w % MACRO == 0

    de1f = de1.reshape(n // STEP, STEP)
    de2f = de2.reshape(n // STEP, STEP)
    ff = f.reshape(n)
    W1p = jnp.pad(W1, ((0, 0), (0, 128 - E)))
    W2p = jnp.pad(W2, ((0, 0), (0, 128 - E)))

    mesh = plsc.VectorSubcoreMesh(core_axis_name="c", subcore_axis_name="s")
    run = pl.kernel(
        functools.partial(_sc_body, rows_per_w),
        out_type=jax.ShapeDtypeStruct((n * OUT_W,), jnp.float32),
        mesh=mesh,
        scratch_types=[
            pltpu.VMEM((NSTEP, STEP), jnp.int32),
            pltpu.VMEM((NSTEP, STEP), jnp.int32),
            pltpu.VMEM((MACRO,), jnp.float32),
            pltpu.VMEM((STEP, 128), jnp.float32),
            pltpu.VMEM((STEP, 128), jnp.float32),
            pltpu.VMEM((STEP, 128), jnp.float32),
            pltpu.VMEM((STEP, 128), jnp.float32),
            pltpu.VMEM((OUT_STEP,), jnp.float32),
            pltpu.VMEM((OUT_STEP,), jnp.float32),
            pltpu.SemaphoreType.DMA,
            pltpu.SemaphoreType.DMA,
            pltpu.SemaphoreType.DMA,
        ],
        compiler_params=pltpu.CompilerParams(needs_layout_passes=False),
    )
    out = run(de1f, de2f, ff, W1p, W2p)
    return out.reshape(B, L, OUT_W)


# speedup vs baseline: 2.8429x; 1.1125x over previous
import functools

import jax
import jax.numpy as jnp
from jax import lax
from jax.experimental import pallas as pl
from jax.experimental.pallas import tpu as pltpu
from jax.experimental.pallas import tpu_sc as plsc

E = 64
OUT_W = 2 * E + 1

STEP = 128
MACRO = 1024
NSTEP = MACRO // STEP
OUT_STEP = STEP * OUT_W


def _sc_body(rows_per_w, de1_hbm, de2_hbm, f_hbm, w1_hbm, w2_hbm,
             out_hbm, idx1_v, idx2_v, f_v, b1_0, b1_1, b2_0, b2_1,
             out_0, out_1, sem1, sem2, sem_o):
    nc = 2
    wid = lax.axis_index("s") * nc + lax.axis_index("c")
    base = wid * rows_per_w
    n_macro = rows_per_w // MACRO
    lanes = lax.iota(jnp.int32, 16)
    b1s, b2s, outs = (b1_0, b1_1), (b2_0, b2_1), (out_0, out_1)

    def gather(j, p):
        pltpu.async_copy(w1_hbm.at[idx1_v.at[j]], b1s[p], sem1)
        pltpu.async_copy(w2_hbm.at[idx2_v.at[j]], b2s[p], sem2)

    def gather_wait(j, p):
        pltpu.make_async_copy(w1_hbm.at[idx1_v.at[j]], b1s[p], sem1).wait()
        pltpu.make_async_copy(w2_hbm.at[idx2_v.at[j]], b2s[p], sem2).wait()

    def out_dst(row0, j):
        return out_hbm.at[pl.ds((row0 + j * STEP) * OUT_W, OUT_STEP)]

    def interleave(j, p):
        b1, b2, out_v = b1s[p], b2s[p], outs[p]

        def row_body(r, addr):
            vals = ([b1[r, pl.ds(16 * k, 16)] for k in range(E // 16)] +
                    [b2[r, pl.ds(16 * k, 16)] for k in range(E // 16)])
            for k in range(2 * (E // 16)):
                plsc.store_scatter(out_v, [addr + 16 * k], vals[k])
            return addr + OUT_W

        lax.fori_loop(0, STEP, row_body, lanes, unroll=2)
        for t in range(STEP // 16):
            addr = (lanes + 16 * t) * OUT_W + 2 * E
            plsc.store_scatter(out_v, [addr],
                               f_v[pl.ds(j * STEP + 16 * t, 16)])

    def macro_body(m, _):
        row0 = pl.multiple_of(base + m * MACRO, MACRO)
        g0 = pl.multiple_of(row0 // STEP, NSTEP)
        pltpu.sync_copy(de1_hbm.at[pl.ds(g0, NSTEP)], idx1_v)
        pltpu.sync_copy(de2_hbm.at[pl.ds(g0, NSTEP)], idx2_v)
        pltpu.sync_copy(f_hbm.at[pl.ds(row0, MACRO)], f_v)
        gather(0, 0)
        for j in range(NSTEP):
            p = j % 2
            if j + 1 < NSTEP:
                gather(j + 1, 1 - p)
            if j >= 2:
                pltpu.make_async_copy(outs[p], out_dst(row0, j - 2),
                                      sem_o).wait()
            gather_wait(j, p)
            interleave(j, p)
            pltpu.async_copy(outs[p], out_dst(row0, j), sem_o)
        for j in (NSTEP - 2, NSTEP - 1):
            pltpu.make_async_copy(outs[j % 2], out_dst(row0, j),
                                  sem_o).wait()

    lax.fori_loop(0, n_macro, macro_body, None)


def kernel(de1, de2, f, W1, W2):
    B, L = de1.shape
    n = B * L
    info = plsc.get_sparse_core_info()
    nw = info.num_cores * info.num_subcores
    rows_per_w = n // nw
    assert rows_per_w % MACRO == 0

    de1f = de1.reshape(n // STEP, STEP)
    de2f = de2.reshape(n // STEP, STEP)
    ff = f.reshape(n)
    W1p = jnp.pad(W1, ((0, 0), (0, 128 - E)))
    W2p = jnp.pad(W2, ((0, 0), (0, 128 - E)))

    mesh = plsc.VectorSubcoreMesh(core_axis_name="c", subcore_axis_name="s")
    run = pl.kernel(
        functools.partial(_sc_body, rows_per_w),
        out_type=jax.ShapeDtypeStruct((n * OUT_W,), jnp.float32),
        mesh=mesh,
        scratch_types=[
            pltpu.VMEM((NSTEP, STEP), jnp.int32),
            pltpu.VMEM((NSTEP, STEP), jnp.int32),
            pltpu.VMEM((MACRO,), jnp.float32),
            pltpu.VMEM((STEP, 128), jnp.float32),
            pltpu.VMEM((STEP, 128), jnp.float32),
            pltpu.VMEM((STEP, 128), jnp.float32),
            pltpu.VMEM((STEP, 128), jnp.float32),
            pltpu.VMEM((OUT_STEP,), jnp.float32),
            pltpu.VMEM((OUT_STEP,), jnp.float32),
            pltpu.SemaphoreType.DMA,
            pltpu.SemaphoreType.DMA,
            pltpu.SemaphoreType.DMA,
        ],
        compiler_params=pltpu.CompilerParams(needs_layout_passes=False),
    )
    out = run(de1f, de2f, ff, W1p, W2p)
    return out.reshape(B, L, OUT_W)
